# baseline (device time: 12314 ns/iter reference)
import jax
import jax.numpy as jnp
from jax import lax
from jax.experimental import pallas as pl
from jax.experimental.pallas import tpu as pltpu

N_Z = 4


def kernel(x, dy, gamma):
    m, d = x.shape

    def body(x_ref, dy_ref, gamma_ref, out_ref, comm_ref, send_sems, recv_sems):
        my_x = lax.axis_index("x")
        my_y = lax.axis_index("y")
        my_z = lax.axis_index("z")

        barrier_sem = pltpu.get_barrier_semaphore()
        for off in (1, 2, 3):
            pl.semaphore_signal(
                barrier_sem,
                inc=1,
                device_id=(my_x, my_y, (my_z + off) % N_Z),
                device_id_type=pl.DeviceIdType.MESH,
            )

        xv = x_ref[...].astype(jnp.float32)
        dyv = dy_ref[...].astype(jnp.float32)
        sx = jnp.sum(xv, axis=1, keepdims=True)
        sxx = jnp.sum(xv * xv, axis=1, keepdims=True)
        mu = sx * (1.0 / d)
        var = sxx * (1.0 / d) - mu * mu
        rstd = lax.rsqrt(var + 1e-5)
        xhat = (xv - mu) * rstd
        dgamma = jnp.sum(dyv * xhat, axis=0)
        dbeta = jnp.sum(dyv, axis=0)
        comm_ref[0] = jnp.stack([dgamma, dbeta])

        pl.semaphore_wait(barrier_sem, 3)

        rdmas = []
        for off in (1, 2, 3):
            slot = N_Z - off
            rdma = pltpu.make_async_remote_copy(
                src_ref=comm_ref.at[0],
                dst_ref=comm_ref.at[slot],
                send_sem=send_sems.at[off - 1],
                recv_sem=recv_sems.at[slot - 1],
                device_id=(my_x, my_y, (my_z + off) % N_Z),
                device_id_type=pl.DeviceIdType.MESH,
            )
            rdma.start()
            rdmas.append(rdma)
        for rdma in rdmas:
            rdma.wait()

        out_ref[...] = (
            comm_ref[0] + comm_ref[1] + comm_ref[2] + comm_ref[3]
        )

    return pl.pallas_call(
        body,
        out_shape=jax.ShapeDtypeStruct((2, d), jnp.float32),
        in_specs=[
            pl.BlockSpec(memory_space=pltpu.VMEM),
            pl.BlockSpec(memory_space=pltpu.VMEM),
            pl.BlockSpec(memory_space=pltpu.VMEM),
        ],
        out_specs=pl.BlockSpec(memory_space=pltpu.VMEM),
        scratch_shapes=[
            pltpu.VMEM((N_Z, 2, d), jnp.float32),
            pltpu.SemaphoreType.DMA((3,)),
            pltpu.SemaphoreType.DMA((3,)),
        ],
        compiler_params=pltpu.CompilerParams(collective_id=0),
    )(x, dy, gamma)
